# baseline (speedup 1.0000x reference)
import functools

import jax
import jax.numpy as jnp
from jax import lax
from jax.experimental import pallas as pl
from jax.experimental.pallas import tpu as pltpu
from jax.experimental.pallas import tpu_sc as plsc

EMB = 64
BATCH = 16384

_info = plsc.get_sparse_core_info()
NC, NS, L = _info.num_cores, _info.num_subcores, _info.num_lanes
NW = NC * NS
BPW = BATCH // NW
CHUNK = 256
NCH = BPW // CHUNK

_mesh = plsc.VectorSubcoreMesh(core_axis_name="c", subcore_axis_name="s")


@functools.partial(
    pl.kernel,
    mesh=_mesh,
    out_type=jax.ShapeDtypeStruct((BATCH * EMB,), jnp.float32),
    scratch_types=[
        pltpu.VMEM((BPW,), jnp.int32),
        pltpu.VMEM((BPW,), jnp.int32),
        pltpu.VMEM((CHUNK, EMB), jnp.float32),
        pltpu.VMEM((CHUNK, EMB), jnp.float32),
        pltpu.VMEM((CHUNK * EMB,), jnp.float32),
        pltpu.SemaphoreType.DMA,
    ],
)
def _path_encoder(idx_cur_hbm, idx_last_hbm, table_hbm, out_hbm,
                  idx_c_v, idx_l_v, rows_c, rows_l, out_v, sem):
    wid = lax.axis_index("s") * NC + lax.axis_index("c")
    base = wid * BPW
    pltpu.sync_copy(idx_cur_hbm.at[pl.ds(base, BPW)], idx_c_v)
    pltpu.sync_copy(idx_last_hbm.at[pl.ds(base, BPW)], idx_l_v)

    def chunk_body(ch, carry):
        off = pl.multiple_of(ch * CHUNK, CHUNK)

        def fire(g, carry):
            s = pl.multiple_of(g * L, L)
            vals_c = idx_c_v[pl.ds(off + s, L)]
            vals_l = idx_l_v[pl.ds(off + s, L)]
            for j in range(L):
                pltpu.async_copy(
                    table_hbm.at[pl.ds(vals_c[j], 1)],
                    rows_c.at[pl.ds(s + j, 1)], sem)
                pltpu.async_copy(
                    table_hbm.at[pl.ds(vals_l[j], 1)],
                    rows_l.at[pl.ds(s + j, 1)], sem)
            return carry

        lax.fori_loop(0, CHUNK // L, fire, 0)
        pltpu.make_async_copy(table_hbm.at[pl.ds(0, CHUNK)], rows_c, sem).wait()
        pltpu.make_async_copy(table_hbm.at[pl.ds(0, CHUNK)], rows_l, sem).wait()

        def mul(i, carry):
            ib = i * EMB
            for c in range(EMB // L):
                a = rows_c[i, pl.ds(c * L, L)]
                b = rows_l[i, pl.ds(c * L, L)]
                out_v[pl.ds(ib + c * L, L)] = a * b
            return carry

        lax.fori_loop(0, CHUNK, mul, 0)
        pltpu.sync_copy(
            out_v, out_hbm.at[pl.ds((base + off) * EMB, CHUNK * EMB)])
        return carry

    lax.fori_loop(0, NCH, chunk_body, 0)


def kernel(actionList, table):
    idx = actionList.astype(jnp.int32)
    flat = _path_encoder(idx[:, 1], idx[:, 0], table)
    return flat.reshape(BATCH, EMB)


# speedup vs baseline: 1.0379x; 1.0379x over previous
import functools

import jax
import jax.numpy as jnp
from jax import lax
from jax.experimental import pallas as pl
from jax.experimental.pallas import tpu as pltpu
from jax.experimental.pallas import tpu_sc as plsc

EMB = 64
BATCH = 16384

_info = plsc.get_sparse_core_info()
NC, NS, L = _info.num_cores, _info.num_subcores, _info.num_lanes
NW = NC * NS
BPW = BATCH // NW
CHUNK = 256
NCH = BPW // CHUNK

_mesh = plsc.VectorSubcoreMesh(core_axis_name="c", subcore_axis_name="s")


@functools.partial(
    pl.kernel,
    mesh=_mesh,
    out_type=jax.ShapeDtypeStruct((BATCH, EMB), jnp.float32),
    scratch_types=[
        pltpu.VMEM((BPW,), jnp.int32),
        pltpu.VMEM((BPW,), jnp.int32),
        pltpu.VMEM((CHUNK, EMB), jnp.float32),
        pltpu.VMEM((CHUNK, EMB), jnp.float32),
        pltpu.SemaphoreType.DMA,
        pltpu.SemaphoreType.DMA,
        pltpu.SemaphoreType.DMA,
        pltpu.SemaphoreType.DMA,
    ],
)
def _path_encoder(idx_cur_hbm, idx_last_hbm, table_hbm, out_hbm,
                  idx_c_v, idx_l_v, rows_c, rows_l, s0, s1, s2, s3):
    wid = lax.axis_index("s") * NC + lax.axis_index("c")
    base = wid * BPW
    pltpu.sync_copy(idx_cur_hbm.at[pl.ds(base, BPW)], idx_c_v)
    pltpu.sync_copy(idx_last_hbm.at[pl.ds(base, BPW)], idx_l_v)

    def chunk_body(ch, carry):
        off = pl.multiple_of(ch * CHUNK, CHUNK)

        @plsc.parallel_loop(0, CHUNK // L, 1, unroll=2)
        def _fire(g):
            s = pl.multiple_of(g * L, L)
            vals_c = idx_c_v[pl.ds(off + s, L)]
            vals_l = idx_l_v[pl.ds(off + s, L)]
            sems = (s0, s1, s2, s3)
            for j in range(L):
                pltpu.async_copy(
                    table_hbm.at[pl.ds(vals_c[j], 1)],
                    rows_c.at[pl.ds(s + j, 1)], sems[j % 4])
                pltpu.async_copy(
                    table_hbm.at[pl.ds(vals_l[j], 1)],
                    rows_l.at[pl.ds(s + j, 1)], sems[j % 4])
        qtr = CHUNK // 4
        for k, sk in enumerate((s0, s1, s2, s3)):
            pltpu.make_async_copy(
                table_hbm.at[pl.ds(0, qtr)],
                rows_c.at[pl.ds(k * qtr, qtr)], sk).wait()
            pltpu.make_async_copy(
                table_hbm.at[pl.ds(0, qtr)],
                rows_l.at[pl.ds(k * qtr, qtr)], sk).wait()

        def mul(i, carry):
            for c in range(EMB // L):
                a = rows_c[i, pl.ds(c * L, L)]
                b = rows_l[i, pl.ds(c * L, L)]
                rows_c[i, pl.ds(c * L, L)] = a * b
            return carry

        lax.fori_loop(0, CHUNK, mul, 0)
        pltpu.sync_copy(rows_c, out_hbm.at[pl.ds(base + off, CHUNK)])
        return carry

    lax.fori_loop(0, NCH, chunk_body, 0)


def kernel(actionList, table):
    idx = actionList.astype(jnp.int32)
    return _path_encoder(idx[:, 1], idx[:, 0], table)
